# Initial kernel scaffold; baseline (speedup 1.0000x reference)
import jax
import jax.numpy as jnp
from jax.experimental import pallas as pl


def kernel(A_image, B_image, cb, cr, LUT00, LUT01, LUT02, LUT03, LUT8, LUTPGF, LUTCB, LUTCR):
    raise NotImplementedError("write your pallas kernel here")


# speedup vs baseline: 1010.2135x; 1010.2135x over previous
import functools

import jax
import jax.numpy as jnp
from jax import lax
from jax.experimental import pallas as pl
from jax.experimental.pallas import tpu as pltpu
from jax.experimental.pallas import tpu_sc as plsc

B, H, W = 4, 384, 384
DIM4 = 17
LUT4_LEN = DIM4 ** 4
LUT4_PAD = 83536
LUT8_PAD = 320
ROWS_PER_WORKER = 48
HALO_ROWS = ROWS_PER_WORKER + 4
NVREG_PG0 = HALO_ROWS * W // 16
CHUNK_ROWS = 8
NCHUNK = ROWS_PER_WORKER // CHUNK_ROWS

STAGE_OFFS = (
    ((0, 0), (0, 1), (1, 0), (1, 1)),
    ((0, 0), (1, 0), (0, -1), (1, -1)),
    ((0, 0), (0, -1), (-1, 0), (-1, -1)),
    ((0, 0), (-1, 0), (0, 1), (-1, 1)),
)
STAGE_ROWS = ((0, 50), (0, 49), (1, 49), (2, 49))


def _interp_frac(x, n_minus_1, i_max):
    p = x * float(n_minus_1)
    ii = jnp.minimum(p.astype(jnp.int32), i_max)
    return ii, p - ii.astype(jnp.float32)


def _body(a_hbm, b_hbm, cb_hbm, cr_hbm, lut4_hbm, lut8_hbm, lut1_hbm,
          out_hbm, bufa, bufb, lutv, lut8v, lut1v):
    wid = lax.axis_index("s") * 2 + lax.axis_index("c")
    g0 = wid * ROWS_PER_WORKER
    img = lax.shift_right_logical(wid, 3)
    m0 = img * H
    iotaf = lax.iota(jnp.int32, 16).astype(jnp.float32)

    pltpu.sync_copy(lut8_hbm, lut8v)
    pltpu.sync_copy(lut1_hbm, lut1v)

    def load_window(src, dst):
        pltpu.sync_copy(src.at[pl.ds(g0 * W, ROWS_PER_WORKER * W)],
                        dst.at[pl.ds(2 * W, ROWS_PER_WORKER * W)])
        for i in range(2):
            srow = jnp.maximum(g0 - 2 + i, m0)
            pltpu.sync_copy(src.at[pl.ds(srow * W, W)], dst.at[pl.ds(i * W, W)])
        for i in range(2):
            srow = jnp.minimum(g0 + ROWS_PER_WORKER + i, m0 + H - 1)
            pltpu.sync_copy(src.at[pl.ds(srow * W, W)],
                            dst.at[pl.ds((50 + i) * W, W)])

    load_window(a_hbm, bufa)
    load_window(b_hbm, bufb)

    def pg0_body(i, _):
        q = i * 16
        av = bufa[pl.ds(q, 16)]
        bv = bufb[pl.ds(q, 16)]
        ia, fa = _interp_frac(av, 16, 15)
        ib, fb = _interp_frac(bv, 16, 15)
        idx = ia * 17 + ib
        t00 = plsc.load_gather(lut8v, [idx])
        t01 = plsc.load_gather(lut8v, [idx + 1])
        t10 = plsc.load_gather(lut8v, [idx + 17])
        t11 = plsc.load_gather(lut8v, [idx + 18])
        v0 = t00 + fb * (t01 - t00)
        v1 = t10 + fb * (t11 - t10)
        val = v0 + fa * (v1 - v0)
        val = jnp.minimum(jnp.maximum(val, 0.0), 1.0)
        bufa[pl.ds(q, 16)] = val
        return _

    lax.fori_loop(0, NVREG_PG0, pg0_body, None)

    def stage(inref, outref, offs, row_lo, row_hi):
        def row_body(t, _):
            vg = g0 - 2 + t
            bases = []
            for (dy, dx) in offs:
                nbg = jnp.minimum(jnp.maximum(vg + dy, m0), m0 + H - 1)
                bases.append((nbg - g0 + 2) * W)

            def col_body(j, __):
                c0 = j * 16
                idxs, fracs = [], []
                for k, (dy, dx) in enumerate(offs):
                    if dx == 0:
                        x = inref[pl.ds(bases[k] + c0, 16)]
                    else:
                        cf = iotaf + (c0 + dx).astype(jnp.float32)
                        cf = jnp.minimum(jnp.maximum(cf, 0.0), float(W - 1))
                        x = plsc.load_gather(
                            inref, [bases[k] + cf.astype(jnp.int32)])
                    ik, fk = _interp_frac(x, DIM4 - 1, DIM4 - 2)
                    idxs.append(ik)
                    fracs.append(fk)
                lin = ((idxs[0] * 17 + idxs[1]) * 17 + idxs[2]) * 17 + idxs[3]
                f0, f1, f2, f3 = fracs
                e0, e1, e2, e3 = 1.0 - f0, 1.0 - f1, 1.0 - f2, 1.0 - f3
                wa = (e0 * e1, e0 * f1, f0 * e1, f0 * f1)
                wb00, wb01, wb10, wb11 = e2 * e3, e2 * f3, f2 * e3, f2 * f3
                acc = None
                for ci, (c0c, c1c) in enumerate(
                        ((0, 0), (0, 1), (1, 0), (1, 1))):
                    base = lin + (c0c * 4913 + c1c * 289)
                    t00 = plsc.load_gather(lutv, [base])
                    t01 = plsc.load_gather(lutv, [base + 1])
                    t10 = plsc.load_gather(lutv, [base + 17])
                    t11 = plsc.load_gather(lutv, [base + 18])
                    sub = wb00 * t00 + wb01 * t01 + wb10 * t10 + wb11 * t11
                    term = wa[ci] * sub
                    acc = term if acc is None else acc + term
                acc = jnp.minimum(jnp.maximum(acc, 0.0), 1.0)
                outref[pl.ds(t * W + c0, 16)] = acc
                return __

            lax.fori_loop(0, W // 16, col_body, None)
            return _

        lax.fori_loop(row_lo, row_hi, row_body, None)

    bufs = (bufa, bufb)
    for s in range(4):
        pltpu.sync_copy(lut4_hbm.at[pl.ds(s * LUT4_PAD, LUT4_PAD)], lutv)
        lo, hi = STAGE_ROWS[s]
        stage(bufs[s % 2], bufs[(s + 1) % 2], STAGE_OFFS[s], lo, hi + 1)

    NC = CHUNK_ROWS * W
    r_in_img = g0 - m0

    def chunk_body(ch, _):
        row = ch * CHUNK_ROWS
        pltpu.sync_copy(cb_hbm.at[pl.ds((g0 + row) * W, NC)],
                        bufb.at[pl.ds(0, NC)])
        pltpu.sync_copy(cr_hbm.at[pl.ds((g0 + row) * W, NC)],
                        bufb.at[pl.ds(NC, NC)])

        def pix_body(i, __):
            q = i * 16
            x = bufa[pl.ds((2 + row) * W + q, 16)]
            ip, fp = _interp_frac(x, 255, 254)
            p0 = plsc.load_gather(lut1v, [ip])
            p1 = plsc.load_gather(lut1v, [ip + 1])
            pg1 = p0 + fp * (p1 - p0)
            cbv = bufb[pl.ds(q, 16)]
            icb, fcbf = _interp_frac(cbv, 255, 254)
            c0 = plsc.load_gather(lut1v, [icb + 256])
            c1 = plsc.load_gather(lut1v, [icb + 257])
            fcb = c0 + fcbf * (c1 - c0) - 0.5
            crv = bufb[pl.ds(NC + q, 16)]
            icr, fcrf = _interp_frac(crv, 255, 254)
            d0 = plsc.load_gather(lut1v, [icr + 512])
            d1 = plsc.load_gather(lut1v, [icr + 513])
            fcr = d0 + fcrf * (d1 - d0) - 0.5
            bufb[pl.ds(2 * NC + q, 16)] = pg1 + fcr * 1.402
            bufb[pl.ds(3 * NC + q, 16)] = pg1 - fcb * 0.344136 - fcr * 0.714136
            bufb[pl.ds(4 * NC + q, 16)] = pg1 + fcb * 1.772
            return __

        lax.fori_loop(0, NC // 16, pix_body, None)
        for c in range(3):
            dst = ((img * 3 + c) * H + r_in_img + row) * W
            pltpu.sync_copy(bufb.at[pl.ds((2 + c) * NC, NC)],
                            out_hbm.at[pl.ds(dst, NC)])
        return _

    lax.fori_loop(0, NCHUNK, chunk_body, None)


@jax.jit
def kernel(A_image, B_image, cb, cr, LUT00, LUT01, LUT02, LUT03,
           LUT8, LUTPGF, LUTCB, LUTCR):
    a = A_image[:, 0].reshape(-1)
    b = B_image[:, 0].reshape(-1)
    cbf = cb[:, 0].reshape(-1)
    crf = cr[:, 0].reshape(-1)
    lut4 = jnp.concatenate([
        jnp.pad(l.reshape(-1), (0, LUT4_PAD - LUT4_LEN))
        for l in (LUT00, LUT01, LUT02, LUT03)])
    lut8 = jnp.pad(LUT8.reshape(-1), (0, LUT8_PAD - LUT8.size))
    lut1 = jnp.concatenate([LUTPGF, LUTCB, LUTCR])

    mesh = plsc.VectorSubcoreMesh(
        core_axis_name="c", subcore_axis_name="s", num_cores=2,
        num_subcores=16)
    run = pl.kernel(
        _body,
        out_type=jax.ShapeDtypeStruct((B * 3 * H * W,), jnp.float32),
        mesh=mesh,
        compiler_params=pltpu.CompilerParams(needs_layout_passes=False),
        scratch_types=[
            pltpu.VMEM((HALO_ROWS * W,), jnp.float32),
            pltpu.VMEM((HALO_ROWS * W,), jnp.float32),
            pltpu.VMEM((LUT4_PAD,), jnp.float32),
            pltpu.VMEM((LUT8_PAD,), jnp.float32),
            pltpu.VMEM((768,), jnp.float32),
        ],
    )
    out = run(a, b, cbf, crf, lut4, lut8, lut1)
    return out.reshape(B, 3, H, W)


# speedup vs baseline: 1483.5656x; 1.4686x over previous
import functools

import jax
import jax.numpy as jnp
from jax import lax
from jax.experimental import pallas as pl
from jax.experimental.pallas import tpu as pltpu
from jax.experimental.pallas import tpu_sc as plsc

B, H, W = 4, 384, 384
DIM4 = 17
LUT4_LEN = DIM4 ** 4
LUT4_PAD = 83536
LUT8_PAD = 320
ROWS_PER_WORKER = 48
HALO_ROWS = ROWS_PER_WORKER + 4
NVREG_PG0 = HALO_ROWS * W // 16
CHUNK_ROWS = 8
NCHUNK = ROWS_PER_WORKER // CHUNK_ROWS

STAGE_OFFS = (
    ((0, 0), (0, 1), (1, 0), (1, 1)),
    ((0, 0), (1, 0), (0, -1), (1, -1)),
    ((0, 0), (0, -1), (-1, 0), (-1, -1)),
    ((0, 0), (-1, 0), (0, 1), (-1, 1)),
)
STAGE_ROWS = ((0, 50), (0, 49), (1, 49), (2, 49))


def _interp_frac(x, n_minus_1, i_max):
    p = x * float(n_minus_1)
    ii = jnp.minimum(p.astype(jnp.int32), i_max)
    return ii, p - ii.astype(jnp.float32)


def _body(a_hbm, b_hbm, cb_hbm, cr_hbm, lut4_hbm, lut8_hbm, lut1_hbm,
          out_hbm, bufa, bufb, lutv, lut8v, lut1v, sem_in, sem_lut):
    wid = lax.axis_index("s") * 2 + lax.axis_index("c")
    g0 = wid * ROWS_PER_WORKER
    img = lax.shift_right_logical(wid, 3)
    m0 = img * H
    iotaf = lax.iota(jnp.int32, 16).astype(jnp.float32)

    descs = [pltpu.async_copy(lut8_hbm, lut8v, sem_in),
             pltpu.async_copy(lut1_hbm, lut1v, sem_in)]

    def load_window(src, dst):
        descs.append(pltpu.async_copy(
            src.at[pl.ds(g0 * W, ROWS_PER_WORKER * W)],
            dst.at[pl.ds(2 * W, ROWS_PER_WORKER * W)], sem_in))
        for i in range(2):
            srow = jnp.maximum(g0 - 2 + i, m0)
            descs.append(pltpu.async_copy(
                src.at[pl.ds(srow * W, W)], dst.at[pl.ds(i * W, W)], sem_in))
        for i in range(2):
            srow = jnp.minimum(g0 + ROWS_PER_WORKER + i, m0 + H - 1)
            descs.append(pltpu.async_copy(
                src.at[pl.ds(srow * W, W)],
                dst.at[pl.ds((50 + i) * W, W)], sem_in))

    load_window(a_hbm, bufa)
    load_window(b_hbm, bufb)
    lut_desc = pltpu.async_copy(lut4_hbm.at[pl.ds(0, LUT4_PAD)], lutv, sem_lut)
    for d in descs:
        d.wait()

    @plsc.parallel_loop(0, NVREG_PG0, unroll=2)
    def pg0_body(i):
        q = i * 16
        av = bufa[pl.ds(q, 16)]
        bv = bufb[pl.ds(q, 16)]
        ia, fa = _interp_frac(av, 16, 15)
        ib, fb = _interp_frac(bv, 16, 15)
        idx = ia * 17 + ib
        t00 = plsc.load_gather(lut8v, [idx])
        t01 = plsc.load_gather(lut8v, [idx + 1])
        t10 = plsc.load_gather(lut8v, [idx + 17])
        t11 = plsc.load_gather(lut8v, [idx + 18])
        v0 = t00 + fb * (t01 - t00)
        v1 = t10 + fb * (t11 - t10)
        val = v0 + fa * (v1 - v0)
        val = jnp.minimum(jnp.maximum(val, 0.0), 1.0)
        bufa[pl.ds(q, 16)] = val

    def stage(inref, outref, offs, row_lo, row_hi):
        def row_body(t, _):
            vg = g0 - 2 + t
            bases = []
            for (dy, dx) in offs:
                nbg = jnp.minimum(jnp.maximum(vg + dy, m0), m0 + H - 1)
                bases.append((nbg - g0 + 2) * W)

            @plsc.parallel_loop(0, W // 16, unroll=2)
            def col_body(j):
                c0 = j * 16
                idxs, fracs = [], []
                for k, (dy, dx) in enumerate(offs):
                    if dx == 0:
                        x = inref[pl.ds(bases[k] + c0, 16)]
                    else:
                        cf = iotaf + (c0 + dx).astype(jnp.float32)
                        cf = jnp.minimum(jnp.maximum(cf, 0.0), float(W - 1))
                        x = plsc.load_gather(
                            inref, [bases[k] + cf.astype(jnp.int32)])
                    ik, fk = _interp_frac(x, DIM4 - 1, DIM4 - 2)
                    idxs.append(ik)
                    fracs.append(fk)
                lin = ((idxs[0] * 17 + idxs[1]) * 17 + idxs[2]) * 17 + idxs[3]
                f0, f1, f2, f3 = fracs
                e0, e1, e2, e3 = 1.0 - f0, 1.0 - f1, 1.0 - f2, 1.0 - f3
                wa = (e0 * e1, e0 * f1, f0 * e1, f0 * f1)
                wb00, wb01, wb10, wb11 = e2 * e3, e2 * f3, f2 * e3, f2 * f3
                acc = None
                for ci, (c0c, c1c) in enumerate(
                        ((0, 0), (0, 1), (1, 0), (1, 1))):
                    base = lin + (c0c * 4913 + c1c * 289)
                    t00 = plsc.load_gather(lutv, [base])
                    t01 = plsc.load_gather(lutv, [base + 1])
                    t10 = plsc.load_gather(lutv, [base + 17])
                    t11 = plsc.load_gather(lutv, [base + 18])
                    sub = wb00 * t00 + wb01 * t01 + wb10 * t10 + wb11 * t11
                    term = wa[ci] * sub
                    acc = term if acc is None else acc + term
                acc = jnp.minimum(jnp.maximum(acc, 0.0), 1.0)
                outref[pl.ds(t * W + c0, 16)] = acc

            return _

        lax.fori_loop(row_lo, row_hi, row_body, None)

    bufs = (bufa, bufb)
    for s in range(4):
        lut_desc.wait()
        lo, hi = STAGE_ROWS[s]
        stage(bufs[s % 2], bufs[(s + 1) % 2], STAGE_OFFS[s], lo, hi + 1)
        if s < 3:
            lut_desc = pltpu.async_copy(
                lut4_hbm.at[pl.ds((s + 1) * LUT4_PAD, LUT4_PAD)], lutv,
                sem_lut)

    NC = CHUNK_ROWS * W
    r_in_img = g0 - m0

    def chunk_body(ch, _):
        row = ch * CHUNK_ROWS
        pltpu.sync_copy(cb_hbm.at[pl.ds((g0 + row) * W, NC)],
                        bufb.at[pl.ds(0, NC)])
        pltpu.sync_copy(cr_hbm.at[pl.ds((g0 + row) * W, NC)],
                        bufb.at[pl.ds(NC, NC)])

        @plsc.parallel_loop(0, NC // 16, unroll=2)
        def pix_body(i):
            q = i * 16
            x = bufa[pl.ds((2 + row) * W + q, 16)]
            ip, fp = _interp_frac(x, 255, 254)
            p0 = plsc.load_gather(lut1v, [ip])
            p1 = plsc.load_gather(lut1v, [ip + 1])
            pg1 = p0 + fp * (p1 - p0)
            cbv = bufb[pl.ds(q, 16)]
            icb, fcbf = _interp_frac(cbv, 255, 254)
            c0 = plsc.load_gather(lut1v, [icb + 256])
            c1 = plsc.load_gather(lut1v, [icb + 257])
            fcb = c0 + fcbf * (c1 - c0) - 0.5
            crv = bufb[pl.ds(NC + q, 16)]
            icr, fcrf = _interp_frac(crv, 255, 254)
            d0 = plsc.load_gather(lut1v, [icr + 512])
            d1 = plsc.load_gather(lut1v, [icr + 513])
            fcr = d0 + fcrf * (d1 - d0) - 0.5
            bufb[pl.ds(2 * NC + q, 16)] = pg1 + fcr * 1.402
            bufb[pl.ds(3 * NC + q, 16)] = pg1 - fcb * 0.344136 - fcr * 0.714136
            bufb[pl.ds(4 * NC + q, 16)] = pg1 + fcb * 1.772

        for c in range(3):
            dst = ((img * 3 + c) * H + r_in_img + row) * W
            pltpu.sync_copy(bufb.at[pl.ds((2 + c) * NC, NC)],
                            out_hbm.at[pl.ds(dst, NC)])
        return _

    lax.fori_loop(0, NCHUNK, chunk_body, None)


@jax.jit
def kernel(A_image, B_image, cb, cr, LUT00, LUT01, LUT02, LUT03,
           LUT8, LUTPGF, LUTCB, LUTCR):
    a = A_image[:, 0].reshape(-1)
    b = B_image[:, 0].reshape(-1)
    cbf = cb[:, 0].reshape(-1)
    crf = cr[:, 0].reshape(-1)
    lut4 = jnp.concatenate([
        jnp.pad(l.reshape(-1), (0, LUT4_PAD - LUT4_LEN))
        for l in (LUT00, LUT01, LUT02, LUT03)])
    lut8 = jnp.pad(LUT8.reshape(-1), (0, LUT8_PAD - LUT8.size))
    lut1 = jnp.concatenate([LUTPGF, LUTCB, LUTCR])

    mesh = plsc.VectorSubcoreMesh(
        core_axis_name="c", subcore_axis_name="s", num_cores=2,
        num_subcores=16)
    run = pl.kernel(
        _body,
        out_type=jax.ShapeDtypeStruct((B * 3 * H * W,), jnp.float32),
        mesh=mesh,
        compiler_params=pltpu.CompilerParams(needs_layout_passes=False),
        scratch_types=[
            pltpu.VMEM((HALO_ROWS * W,), jnp.float32),
            pltpu.VMEM((HALO_ROWS * W,), jnp.float32),
            pltpu.VMEM((LUT4_PAD,), jnp.float32),
            pltpu.VMEM((LUT8_PAD,), jnp.float32),
            pltpu.VMEM((768,), jnp.float32),
            pltpu.SemaphoreType.DMA,
            pltpu.SemaphoreType.DMA,
        ],
    )
    out = run(a, b, cbf, crf, lut4, lut8, lut1)
    return out.reshape(B, 3, H, W)
